# baseline (speedup 1.0000x reference)
import functools

import jax
import jax.numpy as jnp
import numpy as np
from jax import lax
from jax.experimental import pallas as pl
from jax.experimental.pallas import tpu as pltpu
from jax.experimental.pallas import tpu_sc as plsc

N = 10000
DEG = 32
FEAT = 128
EMB = 128
NREL = 4
LANES = 16
NGRP = FEAT // LANES
NIDX = NREL * DEG

NWORKERS = 32
CHUNK = 320
BP = NWORKERS * CHUNK
QCH = 80
NBUF = 2


def _reduce_rows(rows, j0, acc_v, b):
    for g0 in range(0, NGRP, 2):
        s0 = rows[j0, pl.ds(g0 * LANES, LANES)]
        s1 = rows[j0, pl.ds((g0 + 1) * LANES, LANES)]
        for j in range(1, DEG):
            s0 = s0 + rows[j0 + j, pl.ds(g0 * LANES, LANES)]
            s1 = s1 + rows[j0 + j, pl.ds((g0 + 1) * LANES, LANES)]
        acc_v[b, pl.ds(g0 * LANES, LANES)] = s0
        acc_v[b, pl.ds((g0 + 1) * LANES, LANES)] = s1


STAGE = 624
TAIL = N - 16 * STAGE


def _sc_body(nodes_hbm, lf_hbm, nall,
             selff, sums,
             nodes_v, nbr_v, acc0_v, acc1_v, table_sp, *rest):
    rows = rest[:NBUF]
    sems = rest[NBUF:2 * NBUF]
    semn = rest[2 * NBUF]
    semt = rest[2 * NBUF + 1]
    semo = rest[2 * NBUF + 2]
    sid = lax.axis_index("s")
    wid = sid * 2 + lax.axis_index("c")
    base = wid * CHUNK
    pltpu.async_copy(lf_hbm.at[pl.ds(sid * STAGE, STAGE)],
                     table_sp.at[pl.ds(sid * STAGE, STAGE)], semt)

    @pl.when(sid == 0)
    def _():
        pltpu.sync_copy(lf_hbm.at[pl.ds(16 * STAGE, TAIL)],
                        table_sp.at[pl.ds(16 * STAGE, TAIL)])

    pltpu.sync_copy(nodes_hbm.at[pl.ds(base, CHUNK)], nodes_v)

    for k in range(CHUNK // QCH):
        pltpu.async_copy(
            lf_hbm.at[nodes_v.at[pl.ds(k * QCH, QCH)]], acc0_v, sems[0])
        pltpu.make_async_copy(
            lf_hbm.at[nodes_v.at[pl.ds(k * QCH, QCH)]], acc0_v,
            sems[0]).wait()
        pltpu.sync_copy(acc0_v, selff.at[pl.ds(base + k * QCH, QCH)])

    pltpu.make_async_copy(lf_hbm.at[pl.ds(sid * STAGE, STAGE)],
                          table_sp.at[pl.ds(sid * STAGE, STAGE)],
                          semt).wait()
    plsc.subcore_barrier()

    def seg(rp, carry):
        q = rp // 2
        r0 = (rp % 2) * 2

        def idx(b):
            return nbr_v.at[b, pl.ds(r0 * DEG, 2 * DEG)]

        @pl.when(r0 == 0)
        def _():
            pltpu.async_copy(
                nall.at[nodes_v.at[pl.ds(q * QCH, QCH)]], nbr_v, semn)
            pltpu.make_async_copy(
                nall.at[nodes_v.at[pl.ds(q * QCH, QCH)]], nbr_v,
                semn).wait()

        for p in range(NBUF):
            pltpu.async_copy(table_sp.at[idx(p)], rows[p], sems[p])

        @pl.when(rp > 0)
        def _():
            pltpu.make_async_copy(acc0_v, sums.at[pl.ds(base, QCH)],
                                  semo).wait()
            pltpu.make_async_copy(acc1_v, sums.at[pl.ds(base, QCH)],
                                  semo).wait()

        def body(i, carry2):
            b = i * NBUF
            for p in range(NBUF):
                pltpu.make_async_copy(table_sp.at[idx(b + p)],
                                      rows[p], sems[p]).wait()
                _reduce_rows(rows[p], 0, acc0_v, b + p)
                _reduce_rows(rows[p], DEG, acc1_v, b + p)
                nxt = jnp.minimum(b + p + NBUF, QCH - 1)
                pltpu.async_copy(table_sp.at[idx(nxt)], rows[p],
                                 sems[p])
            return carry2

        lax.fori_loop(0, QCH // NBUF, body, 0)
        for p in range(NBUF):
            pltpu.make_async_copy(table_sp.at[idx(0)], rows[p],
                                  sems[p]).wait()
        pltpu.async_copy(acc0_v,
                         sums.at[pl.ds(r0 * BP + base + q * QCH, QCH)],
                         semo)
        pltpu.async_copy(acc1_v,
                         sums.at[pl.ds((r0 + 1) * BP + base + q * QCH,
                                       QCH)], semo)
        return carry

    lax.fori_loop(0, (CHUNK // QCH) * NREL // 2, seg, 0)
    pltpu.make_async_copy(acc0_v, sums.at[pl.ds(base, QCH)], semo).wait()
    pltpu.make_async_copy(acc1_v, sums.at[pl.ds(base, QCH)], semo).wait()


_ROW = jax.ShapeDtypeStruct((BP, FEAT), jnp.float32)

_sc_gather = functools.partial(
    pl.kernel,
    mesh=plsc.VectorSubcoreMesh(core_axis_name="c", subcore_axis_name="s"),
    out_type=[_ROW, jax.ShapeDtypeStruct((NREL * BP, FEAT), jnp.float32)],
    scratch_types=[
        pltpu.VMEM((CHUNK,), jnp.int32),
        pltpu.VMEM((QCH, NIDX), jnp.int32),
        pltpu.VMEM((QCH, FEAT), jnp.float32),
        pltpu.VMEM((QCH, FEAT), jnp.float32),
        pltpu.VMEM_SHARED((N, FEAT), jnp.float32),
    ] + [pltpu.VMEM((2 * DEG, FEAT), jnp.float32) for _ in range(NBUF)]
    + [pltpu.SemaphoreType.DMA for _ in range(NBUF + 3)],
)(_sc_body)


BLK = 1024


def _tc_body(self_ref, s0, s1, s2, s3,
             wa0, wa1, wa2, wa3,
             w1s, w10, w11, w12, w13, b1, w2, b2, out_ref):
    h = jnp.dot(self_ref[...], w1s[...], preferred_element_type=jnp.float32)
    scale = jnp.float32(1.0 / DEG)
    for s_ref, wa_ref, w1_ref in ((s0, wa0, w10), (s1, wa1, w11),
                                  (s2, wa2, w12), (s3, wa3, w13)):
        m = jnp.dot(s_ref[...] * scale, wa_ref[...],
                    preferred_element_type=jnp.float32)
        m = jnp.maximum(m, 0.0)
        h = h + jnp.dot(m, w1_ref[...], preferred_element_type=jnp.float32)
    h = jnp.tanh(h + b1[...])
    out_ref[...] = jnp.dot(h, w2[...],
                           preferred_element_type=jnp.float32) + b2[...]


def _row_spec():
    return pl.BlockSpec((BLK, FEAT), lambda i: (i, 0))


def _full_spec(shape):
    return pl.BlockSpec(shape, lambda i: (0, 0))


def _rel_spec(r):
    return pl.BlockSpec((BLK, FEAT), lambda i, rr=r: (rr * (BP // BLK) + i, 0))


_tc_call = pl.pallas_call(
    _tc_body,
    grid=(BP // BLK,),
    in_specs=[_row_spec()] + [_rel_spec(r) for r in range(4)]
    + [_full_spec((FEAT, EMB)) for _ in range(4)]
    + [_full_spec((FEAT, FEAT))]
    + [_full_spec((EMB, FEAT)) for _ in range(4)]
    + [_full_spec((1, FEAT))]
    + [_full_spec((FEAT, EMB))]
    + [_full_spec((1, EMB))],
    out_specs=pl.BlockSpec((BLK, EMB), lambda i: (i, 0)),
    out_shape=jax.ShapeDtypeStruct((N, EMB), jnp.float32),
)


def kernel(nodes, local_features, neigh0, neigh1, neigh2, neigh3,
           Wa0, Wa1, Wa2, Wa3, W1, b1, W2, b2):
    nodes_p = jnp.pad(nodes.astype(jnp.int32), (0, BP - N))
    nall = jnp.concatenate(
        [neigh0.astype(jnp.int32), neigh1.astype(jnp.int32),
         neigh2.astype(jnp.int32), neigh3.astype(jnp.int32)], axis=1)
    selff, sums = _sc_gather(nodes_p, local_features, nall)
    out = _tc_call(
        selff, sums, sums, sums, sums,
        Wa0, Wa1, Wa2, Wa3,
        W1[:FEAT], W1[FEAT:FEAT + EMB], W1[FEAT + EMB:FEAT + 2 * EMB],
        W1[FEAT + 2 * EMB:FEAT + 3 * EMB], W1[FEAT + 3 * EMB:],
        b1.reshape(1, FEAT), W2, b2.reshape(1, EMB))
    return out


# speedup vs baseline: 1.0387x; 1.0387x over previous
import functools

import jax
import jax.numpy as jnp
import numpy as np
from jax import lax
from jax.experimental import pallas as pl
from jax.experimental.pallas import tpu as pltpu
from jax.experimental.pallas import tpu_sc as plsc

N = 10000
DEG = 32
FEAT = 128
EMB = 128
NREL = 4
LANES = 16
NGRP = FEAT // LANES
NIDX = NREL * DEG

NWORKERS = 32
CHUNK = 320
BP = NWORKERS * CHUNK
QCH = 80
NBUF = 5


def _reduce_rows(rows, acc_v, b):
    for g0 in range(0, NGRP, 2):
        s0 = rows[0, pl.ds(g0 * LANES, LANES)]
        s1 = rows[0, pl.ds((g0 + 1) * LANES, LANES)]
        for j in range(1, DEG):
            s0 = s0 + rows[j, pl.ds(g0 * LANES, LANES)]
            s1 = s1 + rows[j, pl.ds((g0 + 1) * LANES, LANES)]
        acc_v[b, pl.ds(g0 * LANES, LANES)] = s0
        acc_v[b, pl.ds((g0 + 1) * LANES, LANES)] = s1


STAGE = 624
TAIL = N - 16 * STAGE


def _sc_body(nodes_hbm, lf_hbm, nall,
             selff, sums,
             nodes_v, nbr_v, acc_v, table_sp, *rest):
    rows = rest[:NBUF]
    sems = rest[NBUF:2 * NBUF]
    semn = rest[2 * NBUF]
    semt = rest[2 * NBUF + 1]
    semo = rest[2 * NBUF + 2]
    sid = lax.axis_index("s")
    wid = sid * 2 + lax.axis_index("c")
    base = wid * CHUNK
    pltpu.async_copy(lf_hbm.at[pl.ds(sid * STAGE, STAGE)],
                     table_sp.at[pl.ds(sid * STAGE, STAGE)], semt)

    @pl.when(sid == 0)
    def _():
        pltpu.sync_copy(lf_hbm.at[pl.ds(16 * STAGE, TAIL)],
                        table_sp.at[pl.ds(16 * STAGE, TAIL)])

    pltpu.sync_copy(nodes_hbm.at[pl.ds(base, CHUNK)], nodes_v)

    for k in range(CHUNK // QCH):
        pltpu.async_copy(
            lf_hbm.at[nodes_v.at[pl.ds(k * QCH, QCH)]], acc_v, sems[0])
        pltpu.make_async_copy(
            lf_hbm.at[nodes_v.at[pl.ds(k * QCH, QCH)]], acc_v,
            sems[0]).wait()
        pltpu.sync_copy(acc_v, selff.at[pl.ds(base + k * QCH, QCH)])

    pltpu.make_async_copy(lf_hbm.at[pl.ds(sid * STAGE, STAGE)],
                          table_sp.at[pl.ds(sid * STAGE, STAGE)],
                          semt).wait()
    plsc.subcore_barrier()

    def seg(rp, carry):
        q = rp // NREL
        r = rp % NREL

        def idx(b):
            return nbr_v.at[b, pl.ds(r * DEG, DEG)]

        @pl.when(r == 0)
        def _():
            pltpu.async_copy(
                nall.at[nodes_v.at[pl.ds(q * QCH, QCH)]], nbr_v, semn)
            pltpu.make_async_copy(
                nall.at[nodes_v.at[pl.ds(q * QCH, QCH)]], nbr_v,
                semn).wait()

        for p in range(NBUF):
            pltpu.async_copy(table_sp.at[idx(p)], rows[p], sems[p])

        @pl.when(rp > 0)
        def _():
            pltpu.make_async_copy(acc_v, sums.at[pl.ds(base, QCH)],
                                  semo).wait()

        def body(i, carry2):
            b = i * NBUF
            for p in range(NBUF):
                pltpu.make_async_copy(table_sp.at[idx(b + p)],
                                      rows[p], sems[p]).wait()
                _reduce_rows(rows[p], acc_v, b + p)
                nxt = jnp.minimum(b + p + NBUF, QCH - 1)
                pltpu.async_copy(table_sp.at[idx(nxt)], rows[p],
                                 sems[p])
            return carry2

        lax.fori_loop(0, QCH // NBUF, body, 0)
        for p in range(NBUF):
            pltpu.make_async_copy(table_sp.at[idx(0)], rows[p],
                                  sems[p]).wait()
        pltpu.async_copy(acc_v,
                         sums.at[pl.ds(r * BP + base + q * QCH, QCH)],
                         semo)
        return carry

    lax.fori_loop(0, (CHUNK // QCH) * NREL, seg, 0)
    pltpu.make_async_copy(acc_v, sums.at[pl.ds(base, QCH)], semo).wait()


_ROW = jax.ShapeDtypeStruct((BP, FEAT), jnp.float32)

_sc_gather = functools.partial(
    pl.kernel,
    mesh=plsc.VectorSubcoreMesh(core_axis_name="c", subcore_axis_name="s"),
    out_type=[_ROW, jax.ShapeDtypeStruct((NREL * BP, FEAT), jnp.float32)],
    scratch_types=[
        pltpu.VMEM((CHUNK,), jnp.int32),
        pltpu.VMEM((QCH, NIDX), jnp.int32),
        pltpu.VMEM((QCH, FEAT), jnp.float32),
        pltpu.VMEM_SHARED((N, FEAT), jnp.float32),
    ] + [pltpu.VMEM((DEG, FEAT), jnp.float32) for _ in range(NBUF)]
    + [pltpu.SemaphoreType.DMA for _ in range(NBUF + 3)],
)(_sc_body)


BLK = 1024


def _tc_body(self_ref, s0, s1, s2, s3,
             wa0, wa1, wa2, wa3,
             w1s, w10, w11, w12, w13, b1, w2, b2, out_ref):
    h = jnp.dot(self_ref[...], w1s[...], preferred_element_type=jnp.float32)
    scale = jnp.float32(1.0 / DEG)
    for s_ref, wa_ref, w1_ref in ((s0, wa0, w10), (s1, wa1, w11),
                                  (s2, wa2, w12), (s3, wa3, w13)):
        m = jnp.dot(s_ref[...] * scale, wa_ref[...],
                    preferred_element_type=jnp.float32)
        m = jnp.maximum(m, 0.0)
        h = h + jnp.dot(m, w1_ref[...], preferred_element_type=jnp.float32)
    h = jnp.tanh(h + b1[...])
    out_ref[...] = jnp.dot(h, w2[...],
                           preferred_element_type=jnp.float32) + b2[...]


def _row_spec():
    return pl.BlockSpec((BLK, FEAT), lambda i: (i, 0))


def _full_spec(shape):
    return pl.BlockSpec(shape, lambda i: (0, 0))


def _rel_spec(r):
    return pl.BlockSpec((BLK, FEAT), lambda i, rr=r: (rr * (BP // BLK) + i, 0))


_tc_call = pl.pallas_call(
    _tc_body,
    grid=(BP // BLK,),
    in_specs=[_row_spec()] + [_rel_spec(r) for r in range(4)]
    + [_full_spec((FEAT, EMB)) for _ in range(4)]
    + [_full_spec((FEAT, FEAT))]
    + [_full_spec((EMB, FEAT)) for _ in range(4)]
    + [_full_spec((1, FEAT))]
    + [_full_spec((FEAT, EMB))]
    + [_full_spec((1, EMB))],
    out_specs=pl.BlockSpec((BLK, EMB), lambda i: (i, 0)),
    out_shape=jax.ShapeDtypeStruct((N, EMB), jnp.float32),
)


def kernel(nodes, local_features, neigh0, neigh1, neigh2, neigh3,
           Wa0, Wa1, Wa2, Wa3, W1, b1, W2, b2):
    nodes_p = jnp.pad(nodes.astype(jnp.int32), (0, BP - N))
    nall = jnp.concatenate(
        [neigh0.astype(jnp.int32), neigh1.astype(jnp.int32),
         neigh2.astype(jnp.int32), neigh3.astype(jnp.int32)], axis=1)
    selff, sums = _sc_gather(nodes_p, local_features, nall)
    out = _tc_call(
        selff, sums, sums, sums, sums,
        Wa0, Wa1, Wa2, Wa3,
        W1[:FEAT], W1[FEAT:FEAT + EMB], W1[FEAT + EMB:FEAT + 2 * EMB],
        W1[FEAT + 2 * EMB:FEAT + 3 * EMB], W1[FEAT + 3 * EMB:],
        b1.reshape(1, FEAT), W2, b2.reshape(1, EMB))
    return out
